# baseline (speedup 1.0000x reference)
import jax
import jax.numpy as jnp
from jax.experimental import pallas as pl
from jax.experimental.pallas import tpu as pltpu

TOKENS = 32768
EMBED = 64
EXPERTS = 4
CHUNK = 16384
NCHUNK = TOKENS // CHUNK
NBUF = 2


def _body(x_hbm, w_ref, zero_hbm, idx_hbm, buf, zbuf, ibuf, sems, zsem, isems):
    def start(c):
        pltpu.make_async_copy(
            x_hbm.at[:, pl.ds(c * CHUNK, CHUNK)],
            buf.at[c % NBUF],
            sems.at[c % NBUF],
        ).start()

    zbuf[...] = jnp.zeros((EXPERTS, TOKENS), jnp.float32)
    zcopy = pltpu.make_async_copy(zbuf, zero_hbm, zsem)
    zcopy.start()

    start(0)
    icopies = []
    for c in range(NCHUNK):
        if c + 1 < NCHUNK:
            start(c + 1)
        pltpu.make_async_copy(
            x_hbm.at[:, pl.ds(c * CHUNK, CHUNK)],
            buf.at[c % NBUF],
            sems.at[c % NBUF],
        ).wait()
        logits = jax.lax.dot_general(
            w_ref[...], buf[c % NBUF],
            dimension_numbers=(((1,), (0,)), ((), ())),
            preferred_element_type=jnp.float32,
        )
        a0 = logits[0, :]
        a1 = logits[1, :]
        a2 = logits[2, :]
        a3 = logits[3, :]
        m01 = jnp.maximum(a0, a1)
        i01 = jnp.where(a1 > a0, 1, 0)
        n01 = jnp.minimum(a0, a1)
        j01 = jnp.where(a1 > a0, 0, 1)
        m23 = jnp.maximum(a2, a3)
        i23 = jnp.where(a3 > a2, 3, 2)
        n23 = jnp.minimum(a2, a3)
        j23 = jnp.where(a3 > a2, 2, 3)
        cond = m23 > m01
        top1 = jnp.where(cond, i23, i01)
        sec01 = jnp.where(m23 > n01, i23, j01)
        sec23 = jnp.where(n23 > m01, j23, i01)
        top2 = jnp.where(cond, sec23, sec01)
        ibuf[0, c % NBUF] = top1
        ibuf[1, c % NBUF] = top2
        icopy = pltpu.make_async_copy(
            ibuf.at[:, c % NBUF],
            idx_hbm.at[:, pl.ds(c * CHUNK, CHUNK)],
            isems.at[c % NBUF],
        )
        icopy.start()
        icopies.append(icopy)
    for icopy in icopies:
        icopy.wait()
    zcopy.wait()


@jax.jit
def kernel(x, W):
    zeros_t, idx_t = pl.pallas_call(
        _body,
        in_specs=[
            pl.BlockSpec(memory_space=pltpu.MemorySpace.HBM),
            pl.BlockSpec((EXPERTS, EMBED), lambda: (0, 0)),
        ],
        out_specs=[
            pl.BlockSpec(memory_space=pltpu.MemorySpace.HBM),
            pl.BlockSpec(memory_space=pltpu.MemorySpace.HBM),
        ],
        out_shape=[
            jax.ShapeDtypeStruct((EXPERTS, TOKENS), jnp.float32),
            jax.ShapeDtypeStruct((2, TOKENS), jnp.int32),
        ],
        scratch_shapes=[
            pltpu.VMEM((NBUF, EMBED, CHUNK), jnp.float32),
            pltpu.VMEM((EXPERTS, TOKENS), jnp.float32),
            pltpu.VMEM((2, NBUF, CHUNK), jnp.int32),
            pltpu.SemaphoreType.DMA((NBUF,)),
            pltpu.SemaphoreType.DMA,
            pltpu.SemaphoreType.DMA((NBUF,)),
        ],
    )(pltpu.with_memory_space_constraint(x.T, pltpu.MemorySpace.HBM), W)
    return zeros_t.T, idx_t.T


# speedup vs baseline: 1.0746x; 1.0746x over previous
import jax
import jax.numpy as jnp
from jax.experimental import pallas as pl
from jax.experimental.pallas import tpu as pltpu

TOKENS = 32768
EMBED = 64
EXPERTS = 4
CHUNK = 8192
NCHUNK = TOKENS // CHUNK
NBUF = NCHUNK


def _body(x_hbm, w_ref, zero_ref, idx_ref, buf, sems):
    def copy(c):
        return pltpu.make_async_copy(
            x_hbm.at[:, pl.ds(c * CHUNK, CHUNK)],
            buf.at[c % NBUF],
            sems.at[c % NBUF],
        )

    for c in range(NCHUNK):
        copy(c).start()
    for c in range(NCHUNK):
        copy(c).wait()
        logits = jax.lax.dot_general(
            w_ref[...], buf[c % NBUF],
            dimension_numbers=(((1,), (0,)), ((), ())),
            preferred_element_type=jnp.float32,
        )
        a0 = logits[0, :]
        a1 = logits[1, :]
        a2 = logits[2, :]
        a3 = logits[3, :]
        m01 = jnp.maximum(a0, a1)
        i01 = jnp.where(a1 > a0, 1, 0)
        n01 = jnp.minimum(a0, a1)
        j01 = jnp.where(a1 > a0, 0, 1)
        m23 = jnp.maximum(a2, a3)
        i23 = jnp.where(a3 > a2, 3, 2)
        n23 = jnp.minimum(a2, a3)
        j23 = jnp.where(a3 > a2, 2, 3)
        cond = m23 > m01
        top1 = jnp.where(cond, i23, i01)
        sec01 = jnp.where(m23 > n01, i23, j01)
        sec23 = jnp.where(n23 > m01, j23, i01)
        top2 = jnp.where(cond, sec23, sec01)
        idx_ref[0, pl.ds(c * CHUNK, CHUNK)] = top1
        idx_ref[1, pl.ds(c * CHUNK, CHUNK)] = top2
    zero_ref[...] = jnp.zeros((EXPERTS, TOKENS), jnp.float32)


@jax.jit
def kernel(x, W):
    zeros_t, idx_t = pl.pallas_call(
        _body,
        in_specs=[
            pl.BlockSpec(memory_space=pltpu.MemorySpace.HBM),
            pl.BlockSpec((EXPERTS, EMBED), lambda: (0, 0)),
        ],
        out_specs=[
            pl.BlockSpec((EXPERTS, TOKENS), lambda: (0, 0)),
            pl.BlockSpec((2, TOKENS), lambda: (0, 0)),
        ],
        out_shape=[
            jax.ShapeDtypeStruct((EXPERTS, TOKENS), jnp.float32),
            jax.ShapeDtypeStruct((2, TOKENS), jnp.int32),
        ],
        scratch_shapes=[
            pltpu.VMEM((NBUF, EMBED, CHUNK), jnp.float32),
            pltpu.SemaphoreType.DMA((NBUF,)),
        ],
    )(pltpu.with_memory_space_constraint(x.T, pltpu.MemorySpace.HBM), W)
    return zeros_t.T, idx_t.T


# speedup vs baseline: 1.0830x; 1.0079x over previous
import jax
import jax.numpy as jnp
from jax.experimental import pallas as pl
from jax.experimental.pallas import tpu as pltpu

TOKENS = 32768
EMBED = 64
EXPERTS = 4
CHUNK = 4096
NCHUNK = TOKENS // CHUNK
NBUF = NCHUNK


def _body(x_hbm, w_ref, zero_ref, idx_ref, buf, sems):
    def copy(c):
        return pltpu.make_async_copy(
            x_hbm.at[:, pl.ds(c * CHUNK, CHUNK)],
            buf.at[c % NBUF],
            sems.at[c % NBUF],
        )

    for c in range(NCHUNK):
        copy(c).start()
    for c in range(NCHUNK):
        copy(c).wait()
        logits = jax.lax.dot_general(
            w_ref[...], buf[c % NBUF],
            dimension_numbers=(((1,), (0,)), ((), ())),
            preferred_element_type=jnp.float32,
        )
        a0 = logits[0, :]
        a1 = logits[1, :]
        a2 = logits[2, :]
        a3 = logits[3, :]
        m01 = jnp.maximum(a0, a1)
        i01 = jnp.where(a1 > a0, 1, 0)
        n01 = jnp.minimum(a0, a1)
        j01 = jnp.where(a1 > a0, 0, 1)
        m23 = jnp.maximum(a2, a3)
        i23 = jnp.where(a3 > a2, 3, 2)
        n23 = jnp.minimum(a2, a3)
        j23 = jnp.where(a3 > a2, 2, 3)
        cond = m23 > m01
        top1 = jnp.where(cond, i23, i01)
        sec01 = jnp.where(m23 > n01, i23, j01)
        sec23 = jnp.where(n23 > m01, j23, i01)
        top2 = jnp.where(cond, sec23, sec01)
        idx_ref[0, pl.ds(c * CHUNK, CHUNK)] = top1
        idx_ref[1, pl.ds(c * CHUNK, CHUNK)] = top2
    zero_ref[...] = jnp.zeros((EXPERTS, TOKENS), jnp.float32)


@jax.jit
def kernel(x, W):
    zeros_t, idx_t = pl.pallas_call(
        _body,
        in_specs=[
            pl.BlockSpec(memory_space=pltpu.MemorySpace.HBM),
            pl.BlockSpec((EXPERTS, EMBED), lambda: (0, 0)),
        ],
        out_specs=[
            pl.BlockSpec((EXPERTS, TOKENS), lambda: (0, 0)),
            pl.BlockSpec((2, TOKENS), lambda: (0, 0)),
        ],
        out_shape=[
            jax.ShapeDtypeStruct((EXPERTS, TOKENS), jnp.float32),
            jax.ShapeDtypeStruct((2, TOKENS), jnp.int32),
        ],
        scratch_shapes=[
            pltpu.VMEM((NBUF, EMBED, CHUNK), jnp.float32),
            pltpu.SemaphoreType.DMA((NBUF,)),
        ],
    )(pltpu.with_memory_space_constraint(x.T, pltpu.MemorySpace.HBM), W)
    return zeros_t.T, idx_t.T
